# Initial kernel scaffold; baseline (speedup 1.0000x reference)
import jax
import jax.numpy as jnp
from jax.experimental import pallas as pl


def kernel(x, W, b):
    raise NotImplementedError("write your pallas kernel here")


# speedup vs baseline: 4.7725x; 4.7725x over previous
import functools

import jax
import jax.numpy as jnp
from jax.experimental import pallas as pl
from jax.experimental.pallas import tpu as pltpu

TOKENS = 8192
DIM = 4096
NUM_EXPERTS = 64
TOPK = 8
EPSILON = 1e-06

BT = 512


def _fwd_body(x_ref, w_ref, b_ref, masked_ref, colsum_ref):
    logits = jax.lax.dot_general(
        x_ref[...], w_ref[...],
        dimension_numbers=(((1,), (1,)), ((), ())),
        preferred_element_type=jnp.float32,
    ) + b_ref[...]
    row_max = jnp.max(logits, axis=1, keepdims=True)
    e = jnp.exp(logits - row_max)
    gate = e / jnp.sum(e, axis=1, keepdims=True)
    iota = jax.lax.broadcasted_iota(jnp.int32, logits.shape, 1)
    work = logits
    mask = jnp.zeros(logits.shape, jnp.bool_)
    for _ in range(TOPK):
        mx = jnp.max(work, axis=1, keepdims=True)
        is_max = work == mx
        first = jnp.min(jnp.where(is_max, iota, NUM_EXPERTS), axis=1,
                        keepdims=True)
        sel = iota == first
        mask = jnp.logical_or(mask, sel)
        work = jnp.where(sel, -jnp.inf, work)
    masked = jnp.where(mask, gate, 0.0)
    masked_ref[...] = masked
    part = jnp.sum(masked, axis=0, keepdims=True)

    @pl.when(pl.program_id(0) == 0)
    def _init():
        colsum_ref[...] = part

    @pl.when(pl.program_id(0) != 0)
    def _acc():
        colsum_ref[...] += part


def _norm_body(masked_ref, colsum_ref, out_ref):
    out_ref[...] = masked_ref[...] / (colsum_ref[...] + EPSILON)


@jax.jit
def kernel(x, W, b):
    b2 = b.reshape(1, NUM_EXPERTS)
    nb = TOKENS // BT
    masked, colsum = pl.pallas_call(
        _fwd_body,
        grid=(nb,),
        in_specs=[
            pl.BlockSpec((BT, DIM), lambda i: (i, 0)),
            pl.BlockSpec((NUM_EXPERTS, DIM), lambda i: (0, 0)),
            pl.BlockSpec((1, NUM_EXPERTS), lambda i: (0, 0)),
        ],
        out_specs=[
            pl.BlockSpec((BT, NUM_EXPERTS), lambda i: (i, 0)),
            pl.BlockSpec((1, NUM_EXPERTS), lambda i: (0, 0)),
        ],
        out_shape=[
            jax.ShapeDtypeStruct((TOKENS, NUM_EXPERTS), jnp.float32),
            jax.ShapeDtypeStruct((1, NUM_EXPERTS), jnp.float32),
        ],
        compiler_params=pltpu.CompilerParams(
            dimension_semantics=("arbitrary",),
        ),
    )(x, W, b2)
    out = pl.pallas_call(
        _norm_body,
        in_specs=[
            pl.BlockSpec((TOKENS, NUM_EXPERTS), lambda: (0, 0)),
            pl.BlockSpec((1, NUM_EXPERTS), lambda: (0, 0)),
        ],
        out_specs=pl.BlockSpec((TOKENS, NUM_EXPERTS), lambda: (0, 0)),
        out_shape=jax.ShapeDtypeStruct((TOKENS, NUM_EXPERTS), jnp.float32),
    )(masked, colsum)
    return out


# speedup vs baseline: 5.2832x; 1.1070x over previous
import functools

import jax
import jax.numpy as jnp
from jax.experimental import pallas as pl
from jax.experimental.pallas import tpu as pltpu

TOKENS = 8192
DIM = 4096
NUM_EXPERTS = 64
TOPK = 8
EPSILON = 1e-06

BT = 1024


def _fwd_body(x_ref, w_ref, b_ref, masked_ref, colsum_ref):
    logits = jax.lax.dot_general(
        x_ref[...], w_ref[...],
        dimension_numbers=(((1,), (1,)), ((), ())),
        preferred_element_type=jnp.float32,
    ) + b_ref[...]
    row_max = jnp.max(logits, axis=1, keepdims=True)
    e = jnp.exp(logits - row_max)
    gate = e / jnp.sum(e, axis=1, keepdims=True)
    iota = jax.lax.broadcasted_iota(jnp.int32, logits.shape, 1)
    work = logits
    mask = jnp.zeros(logits.shape, jnp.bool_)
    for _ in range(TOPK):
        mx = jnp.max(work, axis=1, keepdims=True)
        is_max = work == mx
        first = jnp.min(jnp.where(is_max, iota, NUM_EXPERTS), axis=1,
                        keepdims=True)
        sel = iota == first
        mask = jnp.logical_or(mask, sel)
        work = jnp.where(sel, -jnp.inf, work)
    masked = jnp.where(mask, gate, 0.0)
    masked_ref[...] = masked
    part = jnp.sum(masked, axis=0, keepdims=True)

    @pl.when(pl.program_id(0) == 0)
    def _init():
        colsum_ref[...] = part

    @pl.when(pl.program_id(0) != 0)
    def _acc():
        colsum_ref[...] += part


def _norm_body(masked_ref, colsum_ref, out_ref):
    out_ref[...] = masked_ref[...] / (colsum_ref[...] + EPSILON)


@jax.jit
def kernel(x, W, b):
    b2 = b.reshape(1, NUM_EXPERTS)
    nb = TOKENS // BT
    masked, colsum = pl.pallas_call(
        _fwd_body,
        grid=(nb,),
        in_specs=[
            pl.BlockSpec((BT, DIM), lambda i: (i, 0)),
            pl.BlockSpec((NUM_EXPERTS, DIM), lambda i: (0, 0)),
            pl.BlockSpec((1, NUM_EXPERTS), lambda i: (0, 0)),
        ],
        out_specs=[
            pl.BlockSpec((BT, NUM_EXPERTS), lambda i: (i, 0)),
            pl.BlockSpec((1, NUM_EXPERTS), lambda i: (0, 0)),
        ],
        out_shape=[
            jax.ShapeDtypeStruct((TOKENS, NUM_EXPERTS), jnp.float32),
            jax.ShapeDtypeStruct((1, NUM_EXPERTS), jnp.float32),
        ],
        compiler_params=pltpu.CompilerParams(
            dimension_semantics=("arbitrary",),
        ),
    )(x, W, b2)
    out = pl.pallas_call(
        _norm_body,
        in_specs=[
            pl.BlockSpec((TOKENS, NUM_EXPERTS), lambda: (0, 0)),
            pl.BlockSpec((1, NUM_EXPERTS), lambda: (0, 0)),
        ],
        out_specs=pl.BlockSpec((TOKENS, NUM_EXPERTS), lambda: (0, 0)),
        out_shape=jax.ShapeDtypeStruct((TOKENS, NUM_EXPERTS), jnp.float32),
    )(masked, colsum)
    return out
